# baseline (speedup 1.0000x reference)
import functools

import jax
import jax.numpy as jnp
from jax.experimental import pallas as pl
from jax.experimental.pallas import tpu as pltpu


def _dot(a, b, precision=jax.lax.Precision.HIGHEST):
    return jax.lax.dot_general(a, b, (((1,), (0,)), ((), ())),
                               preferred_element_type=jnp.float32,
                               precision=precision)


def _softmax(z):
    m = jnp.max(z, axis=1, keepdims=True)
    e = jnp.exp(z - m)
    return e / jnp.sum(e, axis=1, keepdims=True)


def _body(SB, WIN, C, MD, D, offs_ref, xa_ref,
          lor_ref, hir_ref, loc_ref, hic_ref,
          Ws_ref, bs_ref, pe_ref, Wb_ref, F_ref, bb_ref, W2_ref, b2_ref,
          out_ref, scr_ref, sem):
    g = pl.program_id(0)
    start = offs_ref[g * SB]
    astart = (start // 8) * 8

    cp = pltpu.make_async_copy(xa_ref.at[pl.ds(astart, WIN), :], scr_ref, sem)
    cp.start()
    cp.wait()
    xw = scr_ref[:, :D]
    confw = scr_ref[:, D:D + 1]
    predw = scr_ref[:, D + 1:D + 2]

    lor = lor_ref[...].reshape(1, SB)
    hir = hir_ref[...].reshape(1, SB)
    loc = loc_ref[...].reshape(SB, 1)
    hic = hic_ref[...].reshape(SB, 1)

    gidc_i = jax.lax.broadcasted_iota(jnp.int32, (WIN, 1), 0) + astart
    gidc = gidc_i.astype(jnp.float32)
    gidr_i = jax.lax.broadcasted_iota(jnp.int32, (1, WIN), 1) + astart
    gidr = gidr_i.astype(jnp.float32)

    ohS = ((gidc >= lor) & (gidc < hir)).astype(jnp.float32)
    ohST = ((gidr >= loc) & (gidr < hic)).astype(jnp.float32)

    oh9 = (predw == jax.lax.broadcasted_iota(jnp.int32, (1, C), 1
                                             ).astype(jnp.float32))
    emb = _dot(oh9.astype(jnp.float32), pe_ref[...])
    h = jnp.maximum(_dot(xw + emb, Ws_ref[...]) + bs_ref[...], 0.0) * confw

    call = _dot(h, Wb_ref[...])
    offrow = jnp.sum(ohS * lor, axis=1, keepdims=True)
    slot = gidc - offrow
    lanegrp = (jax.lax.broadcasted_iota(jnp.int32, (1, MD * C), 1) // C
               ).astype(jnp.float32)
    sme = (slot == lanegrp).astype(jnp.float32)
    c9 = _dot(call * sme, F_ref[...])

    logits = _dot(ohST, c9) + bb_ref[...]
    p = _softmax(logits)
    o2 = _dot(p, W2_ref[...]) + b2_ref[...]
    out_ref[...] = _softmax(o2)


@jax.jit
def kernel(x, pred, box_len, conf, Ws, bs, pred_emb, W_bbox, b_bbox, W2, b2):
    total, D = x.shape
    B = box_len.shape[0]
    C = W2.shape[0]
    MD = W_bbox.shape[0] // D

    SB = 128
    while B % SB:
        SB //= 2
    G = B // SB
    WIN = SB * MD + 8

    off = jnp.concatenate([jnp.zeros((1,), jnp.int32),
                           jnp.cumsum(box_len.astype(jnp.int32))])
    lo = off[:-1].astype(jnp.float32)
    hi = off[1:].astype(jnp.float32)
    lo_row = lo.reshape(G, 1, SB)
    hi_row = hi.reshape(G, 1, SB)
    lo_col = lo.reshape(G, SB, 1)
    hi_col = hi.reshape(G, SB, 1)

    Tpad = ((total + WIN + 7) // 8) * 8
    xa = jnp.concatenate(
        [x, conf[:, None], pred.astype(jnp.float32)[:, None]], axis=1)
    xa = jnp.pad(xa, ((0, Tpad - total), (0, 0)))

    Wb2 = W_bbox.reshape(MD, D, C).transpose(1, 0, 2).reshape(D, MD * C)
    F = (jnp.arange(MD * C)[:, None] % C == jnp.arange(C)[None, :]
         ).astype(jnp.float32)
    bs2 = bs.reshape(1, D)
    bb2 = b_bbox.reshape(1, C)
    b22 = b2.reshape(1, C)

    grid_spec = pltpu.PrefetchScalarGridSpec(
        num_scalar_prefetch=1,
        grid=(G,),
        scratch_shapes=[pltpu.VMEM((WIN, D + 2), jnp.float32),
                        pltpu.SemaphoreType.DMA],
        in_specs=[
            pl.BlockSpec(memory_space=pltpu.MemorySpace.HBM),
            pl.BlockSpec((1, 1, SB), lambda g, offs: (g, 0, 0)),
            pl.BlockSpec((1, 1, SB), lambda g, offs: (g, 0, 0)),
            pl.BlockSpec((1, SB, 1), lambda g, offs: (g, 0, 0)),
            pl.BlockSpec((1, SB, 1), lambda g, offs: (g, 0, 0)),
            pl.BlockSpec((D, D), lambda g, offs: (0, 0)),
            pl.BlockSpec((1, D), lambda g, offs: (0, 0)),
            pl.BlockSpec((C, D), lambda g, offs: (0, 0)),
            pl.BlockSpec((D, MD * C), lambda g, offs: (0, 0)),
            pl.BlockSpec((MD * C, C), lambda g, offs: (0, 0)),
            pl.BlockSpec((1, C), lambda g, offs: (0, 0)),
            pl.BlockSpec((C, C), lambda g, offs: (0, 0)),
            pl.BlockSpec((1, C), lambda g, offs: (0, 0)),
        ],
        out_specs=pl.BlockSpec((SB, C), lambda g, offs: (g, 0)),
    )

    return pl.pallas_call(
        functools.partial(_body, SB, WIN, C, MD, D),
        grid_spec=grid_spec,
        out_shape=jax.ShapeDtypeStruct((B, C), jnp.float32),
    )(off, xa, lo_row, hi_row, lo_col, hi_col,
      Ws, bs2, pred_emb, Wb2, F, bb2, W2, b22)


# speedup vs baseline: 4.4792x; 4.4792x over previous
import functools

import jax
import jax.numpy as jnp
from jax.experimental import pallas as pl
from jax.experimental.pallas import tpu as pltpu


def _dot(a, b, precision=jax.lax.Precision.DEFAULT):
    return jax.lax.dot_general(a, b, (((1,), (0,)), ((), ())),
                               preferred_element_type=jnp.float32,
                               precision=precision)


def _softmax(z):
    m = jnp.max(z, axis=1, keepdims=True)
    e = jnp.exp(z - m)
    return e / jnp.sum(e, axis=1, keepdims=True)


def _body(SB, WIN, WINX, LASTART, C, MD, D, G, offs_ref, x_ref, cp_ref,
          lor_ref, hir_ref, loc_ref, hic_ref,
          Ws_ref, bs_ref, pe_ref, Wb_ref, F_ref, bb_ref, W2_ref, b2_ref,
          out_ref, xs_ref, cs_ref, semx, semc):
    g = pl.program_id(0)

    def _astart(gg):
        a = (offs_ref[gg * SB] // 8) * 8
        return jnp.minimum(a, LASTART)

    def _copies(gg, buf):
        a = _astart(gg)
        return (pltpu.make_async_copy(x_ref.at[pl.ds(a, WINX), :],
                                      xs_ref.at[buf, pl.ds(0, WINX), :],
                                      semx.at[buf]),
                pltpu.make_async_copy(cp_ref.at[pl.ds(a, WINX), :],
                                      cs_ref.at[buf, pl.ds(0, WINX), :],
                                      semc.at[buf]))

    @pl.when(g == 0)
    def _():
        if WIN > WINX:
            xs_ref[:, WINX:, :] = jnp.zeros((2, WIN - WINX, D), jnp.float32)
            cs_ref[:, WINX:, :] = jnp.zeros((2, WIN - WINX, 2), jnp.float32)
        for c in _copies(0, 0):
            c.start()

    @pl.when(g + 1 < G)
    def _():
        for c in _copies(g + 1, (g + 1) % 2):
            c.start()

    for c in _copies(g, g % 2):
        c.wait()
    astart = _astart(g)
    xw = xs_ref[g % 2]
    cw = cs_ref[g % 2]
    confw = cw[:, 0:1]
    predw = cw[:, 1:2]

    lor = lor_ref[...].reshape(1, SB)
    hir = hir_ref[...].reshape(1, SB)
    loc = loc_ref[...].reshape(SB, 1)
    hic = hic_ref[...].reshape(SB, 1)

    gidc_i = jax.lax.broadcasted_iota(jnp.int32, (WIN, 1), 0) + astart
    gidc = gidc_i.astype(jnp.float32)
    gidr_i = jax.lax.broadcasted_iota(jnp.int32, (1, WIN), 1) + astart
    gidr = gidr_i.astype(jnp.float32)

    ohS = ((gidc >= lor) & (gidc < hir)).astype(jnp.float32)
    ohST = ((gidr >= loc) & (gidr < hic)).astype(jnp.bfloat16)

    oh9 = (predw == jax.lax.broadcasted_iota(jnp.int32, (1, C), 1
                                             ).astype(jnp.float32))
    emb = _dot(oh9.astype(jnp.float32), pe_ref[...])
    h = jnp.maximum(_dot(xw + emb, Ws_ref[...]) + bs_ref[...], 0.0) * confw

    call = _dot(h, Wb_ref[...])
    offrow = jnp.sum(ohS * lor, axis=1, keepdims=True)
    slot = gidc - offrow
    lanegrp = (jax.lax.broadcasted_iota(jnp.int32, (1, MD * C), 1) // C
               ).astype(jnp.float32)
    sme = (slot == lanegrp).astype(jnp.float32)
    c9 = _dot(call * sme, F_ref[...]).astype(jnp.bfloat16)

    logits = _dot(ohST, c9) + bb_ref[...]
    p = _softmax(logits)
    o2 = _dot(p, W2_ref[...]) + b2_ref[...]
    out_ref[...] = _softmax(o2)


@jax.jit
def kernel(x, pred, box_len, conf, Ws, bs, pred_emb, W_bbox, b_bbox, W2, b2):
    total, D = x.shape
    B = box_len.shape[0]
    C = W2.shape[0]
    MD = W_bbox.shape[0] // D

    SB = 512
    while B % SB:
        SB //= 2
    G = B // SB
    base = SB * MD + 8
    WINX = min(base + (total - base) % 8, total)
    WIN = ((WINX + 7) // 8) * 8
    LASTART = max(0, total - WINX)

    off = jnp.concatenate([jnp.zeros((1,), jnp.int32),
                           jnp.cumsum(box_len.astype(jnp.int32))])
    lo = off[:-1].astype(jnp.float32)
    hi = off[1:].astype(jnp.float32)
    lo_row = lo.reshape(G, 1, SB)
    hi_row = hi.reshape(G, 1, SB)
    lo_col = lo.reshape(G, SB, 1)
    hi_col = hi.reshape(G, SB, 1)

    cp = jnp.stack([conf, pred.astype(jnp.float32)], axis=1)

    Wb2 = W_bbox.reshape(MD, D, C).transpose(1, 0, 2).reshape(D, MD * C)
    F = (jnp.arange(MD * C)[:, None] % C == jnp.arange(C)[None, :]
         ).astype(jnp.float32)
    bs2 = bs.reshape(1, D)
    bb2 = b_bbox.reshape(1, C)
    b22 = b2.reshape(1, C)

    grid_spec = pltpu.PrefetchScalarGridSpec(
        num_scalar_prefetch=1,
        grid=(G,),
        scratch_shapes=[pltpu.VMEM((2, WIN, D), jnp.float32),
                        pltpu.VMEM((2, WIN, 2), jnp.float32),
                        pltpu.SemaphoreType.DMA((2,)),
                        pltpu.SemaphoreType.DMA((2,))],
        in_specs=[
            pl.BlockSpec(memory_space=pltpu.MemorySpace.HBM),
            pl.BlockSpec(memory_space=pltpu.MemorySpace.HBM),
            pl.BlockSpec((1, 1, SB), lambda g, offs: (g, 0, 0)),
            pl.BlockSpec((1, 1, SB), lambda g, offs: (g, 0, 0)),
            pl.BlockSpec((1, SB, 1), lambda g, offs: (g, 0, 0)),
            pl.BlockSpec((1, SB, 1), lambda g, offs: (g, 0, 0)),
            pl.BlockSpec((D, D), lambda g, offs: (0, 0)),
            pl.BlockSpec((1, D), lambda g, offs: (0, 0)),
            pl.BlockSpec((C, D), lambda g, offs: (0, 0)),
            pl.BlockSpec((D, MD * C), lambda g, offs: (0, 0)),
            pl.BlockSpec((MD * C, C), lambda g, offs: (0, 0)),
            pl.BlockSpec((1, C), lambda g, offs: (0, 0)),
            pl.BlockSpec((C, C), lambda g, offs: (0, 0)),
            pl.BlockSpec((1, C), lambda g, offs: (0, 0)),
        ],
        out_specs=pl.BlockSpec((SB, C), lambda g, offs: (g, 0)),
    )

    return pl.pallas_call(
        functools.partial(_body, SB, WIN, WINX, LASTART, C, MD, D, G),
        grid_spec=grid_spec,
        out_shape=jax.ShapeDtypeStruct((B, C), jnp.float32),
    )(off, x, cp, lo_row, hi_row, lo_col, hi_col,
      Ws, bs2, pred_emb, Wb2, F, bb2, W2, b22)


# speedup vs baseline: 5.3066x; 1.1847x over previous
import functools

import jax
import jax.numpy as jnp
from jax.experimental import pallas as pl
from jax.experimental.pallas import tpu as pltpu


def _dot(a, b, precision=jax.lax.Precision.DEFAULT):
    return jax.lax.dot_general(a, b, (((1,), (0,)), ((), ())),
                               preferred_element_type=jnp.float32,
                               precision=precision)


def _softmax(z):
    m = jnp.max(z, axis=1, keepdims=True)
    e = jnp.exp(z - m)
    return e / jnp.sum(e, axis=1, keepdims=True)


def _body(SB, WIN, WINX, LASTART, C, MD, D, G, offs_ref, x_ref, cp_ref,
          lor_ref, hir_ref, loc_ref, hic_ref,
          Ws_ref, bs_ref, pe_ref, Wb_ref, F_ref, bb_ref, W2_ref, b2_ref,
          out_ref, xs_ref, cs_ref, semx, semc):
    g = pl.program_id(0)

    def _astart(gg):
        a = (offs_ref[gg * SB] // 8) * 8
        return jnp.minimum(a, LASTART)

    def _copies(gg, buf):
        a = _astart(gg)
        return (pltpu.make_async_copy(x_ref.at[pl.ds(a, WINX), :],
                                      xs_ref.at[buf, pl.ds(0, WINX), :],
                                      semx.at[buf]),
                pltpu.make_async_copy(cp_ref.at[pl.ds(a, WINX), :],
                                      cs_ref.at[buf, pl.ds(0, WINX), :],
                                      semc.at[buf]))

    @pl.when(g == 0)
    def _():
        if WIN > WINX:
            xs_ref[:, WINX:, :] = jnp.zeros((2, WIN - WINX, D), jnp.float32)
            cs_ref[:, WINX:, :] = jnp.zeros((2, WIN - WINX, 2), jnp.float32)
        for c in _copies(0, 0):
            c.start()

    @pl.when(g + 1 < G)
    def _():
        for c in _copies(g + 1, (g + 1) % 2):
            c.start()

    for c in _copies(g, g % 2):
        c.wait()
    astart = _astart(g)
    xw = xs_ref[g % 2]
    cw = cs_ref[g % 2]
    confw = cw[:, 0:1]
    predw = cw[:, 1:2]

    lor = lor_ref[...].reshape(1, SB)
    hir = hir_ref[...].reshape(1, SB)
    loc = loc_ref[...].reshape(SB, 1)
    hic = hic_ref[...].reshape(SB, 1)

    gidc_i = jax.lax.broadcasted_iota(jnp.int32, (WIN, 1), 0) + astart
    gidc = gidc_i.astype(jnp.float32)
    gidr_i = jax.lax.broadcasted_iota(jnp.int32, (1, WIN), 1) + astart
    gidr = gidr_i.astype(jnp.float32)

    ohS = ((gidc >= lor) & (gidc < hir)).astype(jnp.float32)
    ohST = ((gidr >= loc) & (gidr < hic)).astype(jnp.bfloat16)

    oh9 = (predw == jax.lax.broadcasted_iota(jnp.int32, (1, C), 1
                                             ).astype(jnp.float32))
    emb = _dot(oh9.astype(jnp.float32), pe_ref[...])
    h = jnp.maximum(_dot(xw + emb, Ws_ref[...]) + bs_ref[...], 0.0) * confw

    call = _dot(h, Wb_ref[...])
    offrow = jnp.sum(ohS * lor, axis=1, keepdims=True)
    slot = gidc - offrow
    lanegrp = (jax.lax.broadcasted_iota(jnp.int32, (1, MD * C), 1) // C
               ).astype(jnp.float32)
    sme = (slot == lanegrp).astype(jnp.float32)
    c9 = _dot(call * sme, F_ref[...]).astype(jnp.bfloat16)

    logits = _dot(ohST, c9) + bb_ref[...]
    p = _softmax(logits)
    o2 = _dot(p, W2_ref[...]) + b2_ref[...]
    out_ref[...] = _softmax(o2)


@jax.jit
def kernel(x, pred, box_len, conf, Ws, bs, pred_emb, W_bbox, b_bbox, W2, b2):
    total, D = x.shape
    B = box_len.shape[0]
    C = W2.shape[0]
    MD = W_bbox.shape[0] // D

    SB = 256
    while B % SB:
        SB //= 2
    G = B // SB
    base = SB * MD + 8
    WINX = min(base + (total - base) % 8, total)
    WIN = ((WINX + 7) // 8) * 8
    LASTART = max(0, total - WINX)

    off = jnp.concatenate([jnp.zeros((1,), jnp.int32),
                           jnp.cumsum(box_len.astype(jnp.int32))])
    lo = off[:-1].astype(jnp.float32)
    hi = off[1:].astype(jnp.float32)
    lo_row = lo.reshape(G, 1, SB)
    hi_row = hi.reshape(G, 1, SB)
    lo_col = lo.reshape(G, SB, 1)
    hi_col = hi.reshape(G, SB, 1)

    cp = jnp.stack([conf, pred.astype(jnp.float32)], axis=1)

    Wb2 = W_bbox.reshape(MD, D, C).transpose(1, 0, 2).reshape(D, MD * C)
    F = (jnp.arange(MD * C)[:, None] % C == jnp.arange(C)[None, :]
         ).astype(jnp.float32)
    bs2 = bs.reshape(1, D)
    bb2 = b_bbox.reshape(1, C)
    b22 = b2.reshape(1, C)

    grid_spec = pltpu.PrefetchScalarGridSpec(
        num_scalar_prefetch=1,
        grid=(G,),
        scratch_shapes=[pltpu.VMEM((2, WIN, D), jnp.float32),
                        pltpu.VMEM((2, WIN, 2), jnp.float32),
                        pltpu.SemaphoreType.DMA((2,)),
                        pltpu.SemaphoreType.DMA((2,))],
        in_specs=[
            pl.BlockSpec(memory_space=pltpu.MemorySpace.HBM),
            pl.BlockSpec(memory_space=pltpu.MemorySpace.HBM),
            pl.BlockSpec((1, 1, SB), lambda g, offs: (g, 0, 0)),
            pl.BlockSpec((1, 1, SB), lambda g, offs: (g, 0, 0)),
            pl.BlockSpec((1, SB, 1), lambda g, offs: (g, 0, 0)),
            pl.BlockSpec((1, SB, 1), lambda g, offs: (g, 0, 0)),
            pl.BlockSpec((D, D), lambda g, offs: (0, 0)),
            pl.BlockSpec((1, D), lambda g, offs: (0, 0)),
            pl.BlockSpec((C, D), lambda g, offs: (0, 0)),
            pl.BlockSpec((D, MD * C), lambda g, offs: (0, 0)),
            pl.BlockSpec((MD * C, C), lambda g, offs: (0, 0)),
            pl.BlockSpec((1, C), lambda g, offs: (0, 0)),
            pl.BlockSpec((C, C), lambda g, offs: (0, 0)),
            pl.BlockSpec((1, C), lambda g, offs: (0, 0)),
        ],
        out_specs=pl.BlockSpec((SB, C), lambda g, offs: (g, 0)),
    )

    return pl.pallas_call(
        functools.partial(_body, SB, WIN, WINX, LASTART, C, MD, D, G),
        grid_spec=grid_spec,
        out_shape=jax.ShapeDtypeStruct((B, C), jnp.float32),
    )(off, x, cp, lo_row, hi_row, lo_col, hi_col,
      Ws, bs2, pred_emb, Wb2, F, bb2, W2, b22)
